# Initial kernel scaffold; baseline (speedup 1.0000x reference)
import jax
import jax.numpy as jnp
from jax.experimental import pallas as pl


def kernel(x, edge_index, W0, b0, W1, b1):
    raise NotImplementedError("write your pallas kernel here")


# speedup vs baseline: 10.9789x; 10.9789x over previous
import functools

import jax
import jax.numpy as jnp
from jax import lax
from jax.experimental import pallas as pl
from jax.experimental.pallas import tpu as pltpu
from jax.experimental.pallas import tpu_sc as plsc

_NC = 2
_NS = 16
_NW = _NC * _NS
_CH = 128
_LANES = 16
_ZR = 32
_BN = 1000


def _mesh():
    return plsc.VectorSubcoreMesh(core_axis_name="c", subcore_axis_name="s")


def _sc_degree(dst_pad, npad, epw, nchunk):
    rps = npad // _NS

    @functools.partial(
        pl.kernel,
        mesh=_mesh(),
        out_type=jax.ShapeDtypeStruct((_NC, npad), jnp.float32),
        scratch_types=[
            pltpu.VMEM((_CH,), jnp.int32),
            pltpu.VMEM((_CH,), jnp.float32),
            pltpu.VMEM((rps,), jnp.float32),
            pltpu.VMEM_SHARED((npad,), jnp.float32),
        ],
    )
    def deg_kernel(dst_hbm, out_hbm, dst_v, ones_v, zbuf_v, deg_sh):
        cid = lax.axis_index("c")
        sid = lax.axis_index("s")
        wid = sid * _NC + cid
        zero16 = jnp.zeros((_LANES,), jnp.float32)
        one16 = jnp.ones((_LANES,), jnp.float32)
        for i in range(_CH // _LANES):
            ones_v[pl.ds(i * _LANES, _LANES)] = one16

        def zbody(i, carry):
            zbuf_v[pl.ds(i * _LANES, _LANES)] = zero16
            return carry

        lax.fori_loop(0, rps // _LANES, zbody, 0)
        pltpu.sync_copy(zbuf_v, deg_sh.at[pl.ds(sid * rps, rps)])
        plsc.subcore_barrier()

        def body(g, carry):
            base = pl.multiple_of(wid * epw + g * _CH, _CH)
            pltpu.sync_copy(dst_hbm.at[pl.ds(base, _CH)], dst_v)
            pltpu.sync_copy(ones_v, deg_sh.at[dst_v], add=True)
            return carry

        lax.fori_loop(0, nchunk, body, 0)
        plsc.subcore_barrier()
        pltpu.sync_copy(
            deg_sh.at[pl.ds(sid * rps, rps)],
            out_hbm.at[cid, pl.ds(sid * rps, rps)],
        )

    return deg_kernel(dst_pad)


def _sc_scatter(h, src_pad, dst_pad, npad, epw, nchunk):
    d = h.shape[1]
    rps = npad // _NS

    @functools.partial(
        pl.kernel,
        mesh=_mesh(),
        out_type=jax.ShapeDtypeStruct((_NC, npad, d), jnp.float32),
        scratch_types=[
            pltpu.VMEM((_CH,), jnp.int32),
            pltpu.VMEM((_CH,), jnp.int32),
            pltpu.VMEM((_CH, d), jnp.float32),
            pltpu.VMEM((_ZR, d), jnp.float32),
            pltpu.VMEM_SHARED((npad, d), jnp.float32),
            pltpu.SemaphoreType.DMA,
        ],
    )
    def scat_kernel(h_hbm, src_hbm, dst_hbm, out_hbm,
                    src_v, dst_v, rows_v, zrows_v, acc_sh, sem):
        cid = lax.axis_index("c")
        sid = lax.axis_index("s")
        wid = sid * _NC + cid
        zero16 = jnp.zeros((_LANES,), jnp.float32)
        for r in range(_ZR):
            for c in range(d // _LANES):
                zrows_v[r, pl.ds(c * _LANES, _LANES)] = zero16

        def zbody(t, carry):
            pltpu.sync_copy(
                zrows_v, acc_sh.at[pl.ds(sid * rps + t * _ZR, _ZR)]
            )
            return carry

        lax.fori_loop(0, rps // _ZR, zbody, 0)
        plsc.subcore_barrier()

        def body(g, carry):
            base = pl.multiple_of(wid * epw + g * _CH, _CH)
            pltpu.sync_copy(src_hbm.at[pl.ds(base, _CH)], src_v)
            pltpu.sync_copy(dst_hbm.at[pl.ds(base, _CH)], dst_v)
            pltpu.async_copy(h_hbm.at[src_v], rows_v, sem).wait()
            pltpu.sync_copy(rows_v, acc_sh.at[dst_v], add=True)
            return carry

        lax.fori_loop(0, nchunk, body, 0)
        plsc.subcore_barrier()
        pltpu.sync_copy(
            acc_sh.at[pl.ds(sid * rps, rps)],
            out_hbm.at[cid, pl.ds(sid * rps, rps)],
        )

    return scat_kernel(h, src_pad, dst_pad)


def _tc_mm_scale(x, w, d0, d1):
    n, d_in = x.shape
    d_out = w.shape[1]

    def body(x_ref, w_ref, d0_ref, d1_ref, o_ref):
        dinv = lax.rsqrt(d0_ref[...] + d1_ref[...] + 1.0)
        h = jnp.dot(x_ref[...], w_ref[...], preferred_element_type=jnp.float32)
        o_ref[...] = h * dinv

    return pl.pallas_call(
        body,
        grid=(n // _BN,),
        in_specs=[
            pl.BlockSpec((_BN, d_in), lambda i: (i, 0)),
            pl.BlockSpec((d_in, d_out), lambda i: (0, 0)),
            pl.BlockSpec((_BN, 1), lambda i: (i, 0)),
            pl.BlockSpec((_BN, 1), lambda i: (i, 0)),
        ],
        out_specs=pl.BlockSpec((_BN, d_out), lambda i: (i, 0)),
        out_shape=jax.ShapeDtypeStruct((n, d_out), jnp.float32),
    )(x, w, d0, d1)


def _tc_mid(p0, p1, hs, w, b, d0, d1):
    n = hs.shape[0]
    d_h = hs.shape[1]
    d_out = w.shape[1]

    def body(p0_ref, p1_ref, hs_ref, w_ref, b_ref, d0_ref, d1_ref, o_ref):
        dinv = lax.rsqrt(d0_ref[...] + d1_ref[...] + 1.0)
        t = p0_ref[...] + p1_ref[...] + hs_ref[...]
        z = jnp.maximum(t * dinv + b_ref[...], 0.0)
        h = jnp.dot(z, w_ref[...], preferred_element_type=jnp.float32)
        o_ref[...] = h * dinv

    return pl.pallas_call(
        body,
        grid=(n // _BN,),
        in_specs=[
            pl.BlockSpec((_BN, d_h), lambda i: (i, 0)),
            pl.BlockSpec((_BN, d_h), lambda i: (i, 0)),
            pl.BlockSpec((_BN, d_h), lambda i: (i, 0)),
            pl.BlockSpec((d_h, d_out), lambda i: (0, 0)),
            pl.BlockSpec((1, d_h), lambda i: (0, 0)),
            pl.BlockSpec((_BN, 1), lambda i: (i, 0)),
            pl.BlockSpec((_BN, 1), lambda i: (i, 0)),
        ],
        out_specs=pl.BlockSpec((_BN, d_out), lambda i: (i, 0)),
        out_shape=jax.ShapeDtypeStruct((n, d_out), jnp.float32),
    )(p0, p1, hs, w, b, d0, d1)


def _tc_final(p0, p1, hs, b, d0, d1):
    n = hs.shape[0]
    d_h = hs.shape[1]

    def body(p0_ref, p1_ref, hs_ref, b_ref, d0_ref, d1_ref, o_ref):
        dinv = lax.rsqrt(d0_ref[...] + d1_ref[...] + 1.0)
        t = p0_ref[...] + p1_ref[...] + hs_ref[...]
        o_ref[...] = jnp.maximum(t * dinv + b_ref[...], 0.0)

    return pl.pallas_call(
        body,
        grid=(n // _BN,),
        in_specs=[
            pl.BlockSpec((_BN, d_h), lambda i: (i, 0)),
            pl.BlockSpec((_BN, d_h), lambda i: (i, 0)),
            pl.BlockSpec((_BN, d_h), lambda i: (i, 0)),
            pl.BlockSpec((1, d_h), lambda i: (0, 0)),
            pl.BlockSpec((_BN, 1), lambda i: (i, 0)),
            pl.BlockSpec((_BN, 1), lambda i: (i, 0)),
        ],
        out_specs=pl.BlockSpec((_BN, d_h), lambda i: (i, 0)),
        out_shape=jax.ShapeDtypeStruct((n, d_h), jnp.float32),
    )(p0, p1, hs, b, d0, d1)


def kernel(x, edge_index, W0, b0, W1, b1):
    n, _ = x.shape
    e = edge_index.shape[1]

    ei = edge_index.astype(jnp.int32)
    src = ei[0]
    dst = ei[1]

    cpw = -(-e // (_NW * _CH))
    epad = _NW * _CH * cpw
    epw = epad // _NW
    npad = (n // 2048 + 1) * 2048

    pad = epad - e
    src_pad = jnp.concatenate([src, jnp.zeros((pad,), jnp.int32)])
    dst_pad = jnp.concatenate([dst, jnp.full((pad,), n, jnp.int32)])

    deg = _sc_degree(dst_pad, npad, epw, cpw)
    d0 = deg[0, :n].reshape(n, 1)
    d1 = deg[1, :n].reshape(n, 1)
    b0r = b0.reshape(1, -1)
    b1r = b1.reshape(1, -1)

    hs0 = _tc_mm_scale(x, W0, d0, d1)
    parts0 = _sc_scatter(hs0, src_pad, dst_pad, npad, epw, cpw)
    hs1 = _tc_mid(parts0[0, :n], parts0[1, :n], hs0, W1, b0r, d0, d1)
    parts1 = _sc_scatter(hs1, src_pad, dst_pad, npad, epw, cpw)
    out = _tc_final(parts1[0, :n], parts1[1, :n], hs1, b1r, d0, d1)
    return out
